# baseline (speedup 1.0000x reference)
import jax
import jax.numpy as jnp
from jax.experimental import pallas as pl
from jax.experimental.pallas import tpu as pltpu

_DIM = 4096
_E = 64
_TOPK = 8
_T = 16384
_BT = 1024
_NB = _T // _BT


def _router_kernel(x_ref, w_ref,
                   idx_ref, wts_ref, psum_ref, fcnt_ref, lse2_ref):
    x = x_ref[...]
    w = w_ref[...]
    logits = jax.lax.dot_general(
        w, x, (((1,), (1,)), ((), ())),
        preferred_element_type=jnp.float32)

    m = jnp.max(logits, axis=0, keepdims=True)
    e = jnp.exp(logits - m)
    s = jnp.sum(e, axis=0, keepdims=True)
    probs = e / s
    lse = m + jnp.log(s)
    lse2_ref[...] = jnp.sum(lse * lse).reshape(1, 1, 1)

    iota_e = jax.lax.broadcasted_iota(jnp.int32, (_E, _BT), 0)
    keys = jax.lax.bitcast_convert_type(probs, jnp.int32)
    vals = []
    idxs = []
    top1 = None
    for k in range(_TOPK):
        kmax = jnp.max(keys, axis=0, keepdims=True)
        idx = jnp.min(jnp.where(keys == kmax, iota_e, _E),
                      axis=0, keepdims=True)
        v = jax.lax.bitcast_convert_type(kmax, jnp.float32)
        vals.append(v)
        idxs.append(idx)
        if k == 0:
            top1 = idx
        keys = jnp.where(iota_e == idx, jnp.int32(-1), keys)

    tv = jnp.concatenate(vals, axis=0)
    ti = jnp.concatenate(idxs, axis=0)
    wts = tv / (jnp.sum(tv, axis=0, keepdims=True) + 1e-8)
    idx_ref[...] = ti.T
    wts_ref[...] = wts.T

    psum_ref[...] = jnp.sum(probs, axis=1, keepdims=True).T.reshape(1, 1, _E)
    fcnt_ref[...] = jnp.sum((top1 == iota_e).astype(jnp.float32),
                            axis=1, keepdims=True).T.reshape(1, 1, _E)


def _finish_kernel(psum_ref, fcnt_ref, lse2_ref, bal_ref, z_ref, usage_ref):
    f = jnp.sum(fcnt_ref[...], axis=0, keepdims=True) / _T
    pmean = jnp.sum(psum_ref[...], axis=0, keepdims=True) / _T
    bal_ref[...] = (_E * jnp.sum(f * pmean)).reshape(1, 1)
    z_ref[...] = (jnp.sum(lse2_ref[...]) / _T).reshape(1, 1)
    usage_ref[...] = f


def kernel(x, W_gate):
    idx, wts, psum, fcnt, lse2 = pl.pallas_call(
        _router_kernel,
        grid=(_NB,),
        in_specs=[
            pl.BlockSpec((_BT, _DIM), lambda i: (i, 0)),
            pl.BlockSpec((_E, _DIM), lambda i: (0, 0)),
        ],
        out_specs=[
            pl.BlockSpec((_BT, _TOPK), lambda i: (i, 0)),
            pl.BlockSpec((_BT, _TOPK), lambda i: (i, 0)),
            pl.BlockSpec((1, 1, _E), lambda i: (i, 0, 0)),
            pl.BlockSpec((1, 1, _E), lambda i: (i, 0, 0)),
            pl.BlockSpec((1, 1, 1), lambda i: (i, 0, 0)),
        ],
        out_shape=[
            jax.ShapeDtypeStruct((_T, _TOPK), jnp.int32),
            jax.ShapeDtypeStruct((_T, _TOPK), jnp.float32),
            jax.ShapeDtypeStruct((_NB, 1, _E), jnp.float32),
            jax.ShapeDtypeStruct((_NB, 1, _E), jnp.float32),
            jax.ShapeDtypeStruct((_NB, 1, 1), jnp.float32),
        ],
        compiler_params=pltpu.CompilerParams(
            dimension_semantics=("parallel",),
        ),
    )(x, W_gate)

    bal, z, usage = pl.pallas_call(
        _finish_kernel,
        out_shape=[
            jax.ShapeDtypeStruct((1, 1), jnp.float32),
            jax.ShapeDtypeStruct((1, 1), jnp.float32),
            jax.ShapeDtypeStruct((1, _E), jnp.float32),
        ],
    )(psum.reshape(_NB, _E), fcnt.reshape(_NB, _E), lse2.reshape(_NB, 1))
    return (idx, wts, bal[0, 0], z[0, 0], usage[0])


# speedup vs baseline: 1.0142x; 1.0142x over previous
import jax
import jax.numpy as jnp
from jax.experimental import pallas as pl
from jax.experimental.pallas import tpu as pltpu

_DIM = 4096
_E = 64
_TOPK = 8
_T = 16384
_BT = 1024
_NB = _T // _BT


def _router_kernel(x_ref, w_ref,
                   idx_ref, wts_ref, bal_ref, z_ref, usage_ref,
                   psum_ref, fcnt_ref, lse2_ref):
    i = pl.program_id(0)

    @pl.when(i == 0)
    def _init():
        psum_ref[...] = jnp.zeros_like(psum_ref)
        fcnt_ref[...] = jnp.zeros_like(fcnt_ref)
        lse2_ref[...] = jnp.zeros_like(lse2_ref)

    x = x_ref[...]
    w = w_ref[...]
    logits = jax.lax.dot_general(
        w, x, (((1,), (1,)), ((), ())),
        preferred_element_type=jnp.float32)

    m = jnp.max(logits, axis=0, keepdims=True)
    e = jnp.exp(logits - m)
    s = jnp.sum(e, axis=0, keepdims=True)
    probs = e / s
    lse = m + jnp.log(s)
    lse2_ref[...] += jnp.sum(lse * lse).reshape(1, 1)

    iota_e = jax.lax.broadcasted_iota(jnp.int32, (_E, _BT), 0)
    keys = jax.lax.bitcast_convert_type(probs, jnp.int32)
    vals = []
    idxs = []
    top1 = None
    for k in range(_TOPK):
        kmax = jnp.max(keys, axis=0, keepdims=True)
        idx = jnp.min(jnp.where(keys == kmax, iota_e, _E),
                      axis=0, keepdims=True)
        v = jax.lax.bitcast_convert_type(kmax, jnp.float32)
        vals.append(v)
        idxs.append(idx)
        if k == 0:
            top1 = idx
        keys = jnp.where(iota_e == idx, jnp.int32(-1), keys)

    tv = jnp.concatenate(vals, axis=0)
    ti = jnp.concatenate(idxs, axis=0)
    wts = tv / (jnp.sum(tv, axis=0, keepdims=True) + 1e-8)
    idx_ref[...] = ti.T
    wts_ref[...] = wts.T

    psum_ref[...] += jnp.sum(probs, axis=1, keepdims=True)
    fcnt_ref[...] += jnp.sum((top1 == iota_e).astype(jnp.float32),
                             axis=1, keepdims=True)

    @pl.when(i == _NB - 1)
    def _fini():
        f = fcnt_ref[...] / _T
        pmean = psum_ref[...] / _T
        bal_ref[...] = (_E * jnp.sum(f * pmean)).reshape(1, 1)
        z_ref[...] = lse2_ref[...] / _T
        usage_ref[...] = f.T


def kernel(x, W_gate):
    idx, wts, bal, z, usage = pl.pallas_call(
        _router_kernel,
        grid=(_NB,),
        in_specs=[
            pl.BlockSpec((_BT, _DIM), lambda i: (i, 0)),
            pl.BlockSpec((_E, _DIM), lambda i: (0, 0)),
        ],
        out_specs=[
            pl.BlockSpec((_BT, _TOPK), lambda i: (i, 0)),
            pl.BlockSpec((_BT, _TOPK), lambda i: (i, 0)),
            pl.BlockSpec((1, 1), lambda i: (0, 0)),
            pl.BlockSpec((1, 1), lambda i: (0, 0)),
            pl.BlockSpec((1, _E), lambda i: (0, 0)),
        ],
        out_shape=[
            jax.ShapeDtypeStruct((_T, _TOPK), jnp.int32),
            jax.ShapeDtypeStruct((_T, _TOPK), jnp.float32),
            jax.ShapeDtypeStruct((1, 1), jnp.float32),
            jax.ShapeDtypeStruct((1, 1), jnp.float32),
            jax.ShapeDtypeStruct((1, _E), jnp.float32),
        ],
        scratch_shapes=[
            pltpu.VMEM((_E, 1), jnp.float32),
            pltpu.VMEM((_E, 1), jnp.float32),
            pltpu.VMEM((1, 1), jnp.float32),
        ],
    )(x, W_gate)
    return (idx, wts, bal[0, 0], z[0, 0], usage[0])


# speedup vs baseline: 1.0149x; 1.0008x over previous
import jax
import jax.numpy as jnp
from jax.experimental import pallas as pl
from jax.experimental.pallas import tpu as pltpu

_DIM = 4096
_E = 64
_TOPK = 8
_T = 16384
_BT = 1024
_NB = _T // _BT
_HD = _DIM // 2


def _router_kernel(x1_ref, x2_ref, w_ref,
                   idx_ref, wts_ref, bal_ref, z_ref, usage_ref,
                   psum_ref, fcnt_ref, lse2_ref):
    i = pl.program_id(0)

    @pl.when(i == 0)
    def _init():
        psum_ref[...] = jnp.zeros_like(psum_ref)
        fcnt_ref[...] = jnp.zeros_like(fcnt_ref)
        lse2_ref[...] = jnp.zeros_like(lse2_ref)

    w = w_ref[...]
    dims = (((1,), (1,)), ((), ()))
    logits = (
        jax.lax.dot_general(w[:, :_HD], x1_ref[...], dims,
                            preferred_element_type=jnp.float32)
        + jax.lax.dot_general(w[:, _HD:], x2_ref[...], dims,
                              preferred_element_type=jnp.float32)
    )

    m = jnp.max(logits, axis=0, keepdims=True)
    e = jnp.exp(logits - m)
    s = jnp.sum(e, axis=0, keepdims=True)
    probs = e / s
    lse = m + jnp.log(s)
    lse2_ref[...] += jnp.sum(lse * lse).reshape(1, 1)

    iota_e = jax.lax.broadcasted_iota(jnp.int32, (_E, _BT), 0)
    keys = jax.lax.bitcast_convert_type(probs, jnp.int32)
    vals = []
    idxs = []
    top1 = None
    for k in range(_TOPK):
        kmax = jnp.max(keys, axis=0, keepdims=True)
        idx = jnp.min(jnp.where(keys == kmax, iota_e, _E),
                      axis=0, keepdims=True)
        v = jax.lax.bitcast_convert_type(kmax, jnp.float32)
        vals.append(v)
        idxs.append(idx)
        if k == 0:
            top1 = idx
        keys = jnp.where(iota_e == idx, jnp.int32(-1), keys)

    tv = jnp.concatenate(vals, axis=0)
    ti = jnp.concatenate(idxs, axis=0)
    wts = tv / (jnp.sum(tv, axis=0, keepdims=True) + 1e-8)
    idx_ref[...] = ti.T
    wts_ref[...] = wts.T

    psum_ref[...] += jnp.sum(probs, axis=1, keepdims=True)
    fcnt_ref[...] += jnp.sum((top1 == iota_e).astype(jnp.float32),
                             axis=1, keepdims=True)

    @pl.when(i == _NB - 1)
    def _fini():
        f = fcnt_ref[...] / _T
        pmean = psum_ref[...] / _T
        bal_ref[...] = (_E * jnp.sum(f * pmean)).reshape(1, 1)
        z_ref[...] = lse2_ref[...] / _T
        usage_ref[...] = f.T


def kernel(x, W_gate):
    idx, wts, bal, z, usage = pl.pallas_call(
        _router_kernel,
        grid=(_NB,),
        in_specs=[
            pl.BlockSpec((_BT, _HD), lambda i: (i, 0)),
            pl.BlockSpec((_BT, _HD), lambda i: (i, 1)),
            pl.BlockSpec((_E, _DIM), lambda i: (0, 0)),
        ],
        out_specs=[
            pl.BlockSpec((_BT, _TOPK), lambda i: (i, 0)),
            pl.BlockSpec((_BT, _TOPK), lambda i: (i, 0)),
            pl.BlockSpec((1, 1), lambda i: (0, 0)),
            pl.BlockSpec((1, 1), lambda i: (0, 0)),
            pl.BlockSpec((1, _E), lambda i: (0, 0)),
        ],
        out_shape=[
            jax.ShapeDtypeStruct((_T, _TOPK), jnp.int32),
            jax.ShapeDtypeStruct((_T, _TOPK), jnp.float32),
            jax.ShapeDtypeStruct((1, 1), jnp.float32),
            jax.ShapeDtypeStruct((1, 1), jnp.float32),
            jax.ShapeDtypeStruct((1, _E), jnp.float32),
        ],
        scratch_shapes=[
            pltpu.VMEM((_E, 1), jnp.float32),
            pltpu.VMEM((_E, 1), jnp.float32),
            pltpu.VMEM((1, 1), jnp.float32),
        ],
    )(x, x, W_gate)
    return (idx, wts, bal[0, 0], z[0, 0], usage[0])
